# baseline (speedup 1.0000x reference)
import jax
import jax.numpy as jnp
import numpy as np
from jax.experimental import pallas as pl

B, C_IN, C_OUT = 4, 256, 128
HF, WF = 28, 28
H, W = 224, 224
NX, NY = 200, 200
GN_GROUPS = 32
ROWS_PER_BLK = 8
N_ROW_BLKS = H // ROWS_PER_BLK
N_PIX = B * H * W
M_BINS = B * NX * NY


def _interp_matrix(n_out: int, n_in: int) -> np.ndarray:
    src = (np.arange(n_out, dtype=np.float64) + 0.5) * (n_in / n_out) - 0.5
    i0 = np.floor(src)
    frac = src - i0
    i0c = np.clip(i0, 0, n_in - 1).astype(np.int64)
    i1c = np.clip(i0 + 1, 0, n_in - 1).astype(np.int64)
    mat = np.zeros((n_out, n_in), dtype=np.float64)
    for y in range(n_out):
        mat[y, i0c[y]] += 1.0 - frac[y]
        mat[y, i1c[y]] += frac[y]
    return mat.astype(np.float32)


_R_UP = jnp.asarray(_interp_matrix(H, HF))
_C_UP = jnp.asarray(_interp_matrix(W, WF))

_GRP = C_OUT // GN_GROUPS
_G_AVG = jnp.asarray(np.kron(np.eye(GN_GROUPS, dtype=np.float32),
                             np.full((_GRP, _GRP), 1.0 / _GRP, np.float32)))


def _conv_gn_silu_kernel(x_ref, w_ref, g_ref, gn_g_ref, gn_b_ref, h_ref):
    hT = jax.lax.dot_general(x_ref[0], w_ref[...], (((0,), (1,)), ((), ())),
                             preferred_element_type=jnp.float32)
    inv_n = 1.0 / (HF * WF)
    mu_c = jnp.sum(hT, axis=0, keepdims=True) * inv_n
    sq_c = jnp.sum(hT * hT, axis=0, keepdims=True) * inv_n
    mu_g = jnp.dot(mu_c, g_ref[...], preferred_element_type=jnp.float32)
    ex2_g = jnp.dot(sq_c, g_ref[...], preferred_element_type=jnp.float32)
    var_g = ex2_g - mu_g * mu_g
    hT = (hT - mu_g) * jax.lax.rsqrt(var_g + 1e-5)
    hT = hT * gn_g_ref[...] + gn_b_ref[...]
    h_ref[0] = hT * jax.nn.sigmoid(hT)


def _row_up_kernel(rup_ref, h_ref, o_ref):
    o_ref[0] = jnp.dot(rup_ref[...], h_ref[0],
                       preferred_element_type=jnp.float32)


def _feat_kernel(h_ref, cup_ref, d_ref, p_ref, lng_ref, lnb_ref, aw_ref,
                 feat_ref, logw_ref, valid_ref, idx_ref):
    blk = pl.program_id(0)
    b = blk // N_ROW_BLKS
    rb = blk - b * N_ROW_BLKS
    p = p_ref[0, 0]
    cup = cup_ref[...]
    for y in range(ROWS_PER_BLK):
        fy = jnp.dot(cup, h_ref[0, y], preferred_element_type=jnp.float32)
        feat_ref[0, y] = fy
        mu = jnp.mean(fy, axis=1, keepdims=True)
        var = jnp.mean((fy - mu) ** 2, axis=1, keepdims=True)
        fn = (fy - mu) * jax.lax.rsqrt(var + 1e-5)
        fn = fn * lng_ref[...] + lnb_ref[...]
        logw = jnp.sum(fn * aw_ref[...], axis=1, keepdims=True) + p[14]
        logw_ref[0, y] = logw * p[15]
        d = d_ref[0, pl.ds(y, 1), :]
        us = jax.lax.broadcasted_iota(jnp.int32, (1, W), 1).astype(jnp.float32)
        vs = (rb * ROWS_PER_BLK + y).astype(jnp.float32)
        xs = (us - p[2]) * d / p[0]
        ys = (vs - p[3]) * d / p[1]
        bf = lambda v: v.astype(jnp.bfloat16).astype(jnp.float32)
        xs_b, ys_b, d_b = bf(xs), bf(ys), bf(d)
        ex = bf(p[4]) * xs_b + bf(p[5]) * ys_b + bf(p[6]) * d_b + p[10]
        ey = bf(p[7]) * xs_b + bf(p[8]) * ys_b + bf(p[9]) * d_b + p[11]
        res = p[12]
        y_min = -(NY * res) / 2.0
        vx = jnp.floor(ex / res).astype(jnp.int32)
        vy = jnp.floor((ey - y_min) / res).astype(jnp.int32)
        valid = (vx >= 0) & (vx < NX) & (vy >= 0) & (vy < NY)
        idx = vx * NY + vy + b * (NX * NY)
        valid_ref[0, pl.ds(y, 1), :] = valid.astype(jnp.float32)
        idx_ref[0, pl.ds(y, 1), :] = jnp.where(valid, idx, 0)


def _weights_kernel(feat_ref, logw_ref, valid_ref, m_ref, wf_ref, w_ref):
    m = m_ref[0, 0]
    lw = jnp.where(valid_ref[...] > 0.0, logw_ref[...] - m, -80.0)
    w = jnp.exp(lw) * valid_ref[...]
    w_ref[...] = w
    wf_ref[...] = feat_ref[...] * w


def _norm_kernel(acc_ref, ws_ref, bev_ref, mask_ref):
    ws = ws_ref[...]
    bev_ref[...] = acc_ref[...] / jnp.clip(ws, 1e-4, None)
    mask_ref[...] = (ws > 1e-6).astype(jnp.float32)


@jax.jit
def kernel(x, depth, K, cam2enu, resolution, conv_w, gn_gamma, gn_beta,
           log_temp, ln_gamma, ln_beta, attn_w, attn_b):
    f32 = jnp.float32
    h = pl.pallas_call(
        _conv_gn_silu_kernel,
        grid=(B,),
        in_specs=[
            pl.BlockSpec((1, C_IN, HF * WF), lambda b: (b, 0, 0)),
            pl.BlockSpec((C_OUT, C_IN), lambda b: (0, 0)),
            pl.BlockSpec((C_OUT, C_OUT), lambda b: (0, 0)),
            pl.BlockSpec((1, C_OUT), lambda b: (0, 0)),
            pl.BlockSpec((1, C_OUT), lambda b: (0, 0)),
        ],
        out_specs=pl.BlockSpec((1, HF * WF, C_OUT), lambda b: (b, 0, 0)),
        out_shape=jax.ShapeDtypeStruct((B, HF * WF, C_OUT), f32),
    )(x.reshape(B, C_IN, HF * WF), conv_w, _G_AVG,
      gn_gamma.reshape(1, C_OUT), gn_beta.reshape(1, C_OUT))

    h2 = h.reshape(B, HF, WF * C_OUT)
    hrow = pl.pallas_call(
        _row_up_kernel,
        grid=(B,),
        in_specs=[
            pl.BlockSpec((H, HF), lambda b: (0, 0)),
            pl.BlockSpec((1, HF, WF * C_OUT), lambda b: (b, 0, 0)),
        ],
        out_specs=pl.BlockSpec((1, H, WF * C_OUT), lambda b: (b, 0, 0)),
        out_shape=jax.ShapeDtypeStruct((B, H, WF * C_OUT), f32),
    )(_R_UP, h2)

    fx, fy = K[:, 0, 0], K[:, 1, 1]
    cx, cy = K[:, 0, 2], K[:, 1, 2]
    Rm = cam2enu[:, :3, :3]
    t = cam2enu[:, :3, 3]
    inv_temp = jnp.exp(-log_temp)
    params = jnp.stack([
        fx, fy, cx, cy,
        Rm[:, 0, 0], Rm[:, 0, 1], Rm[:, 0, 2],
        Rm[:, 1, 0], Rm[:, 1, 1], Rm[:, 1, 2],
        t[:, 0], t[:, 1], resolution,
        jnp.zeros((B,), f32),
        jnp.broadcast_to(attn_b[0], (B,)),
        jnp.broadcast_to(inv_temp, (B,)),
    ], axis=1).reshape(B, 1, 16)

    NB = B * N_ROW_BLKS
    hblk = hrow.reshape(NB, ROWS_PER_BLK, WF, C_OUT)
    feat4, logw4, valid4, idx4 = pl.pallas_call(
        _feat_kernel,
        grid=(NB,),
        in_specs=[
            pl.BlockSpec((1, ROWS_PER_BLK, WF, C_OUT), lambda i: (i, 0, 0, 0)),
            pl.BlockSpec((W, WF), lambda i: (0, 0)),
            pl.BlockSpec((1, ROWS_PER_BLK, W),
                         lambda i: (i // N_ROW_BLKS, i % N_ROW_BLKS, 0)),
            pl.BlockSpec((1, 1, 16), lambda i: (i // N_ROW_BLKS, 0, 0)),
            pl.BlockSpec((1, C_OUT), lambda i: (0, 0)),
            pl.BlockSpec((1, C_OUT), lambda i: (0, 0)),
            pl.BlockSpec((1, C_OUT), lambda i: (0, 0)),
        ],
        out_specs=[
            pl.BlockSpec((1, ROWS_PER_BLK, W, C_OUT), lambda i: (i, 0, 0, 0)),
            pl.BlockSpec((1, ROWS_PER_BLK, W, 1), lambda i: (i, 0, 0, 0)),
            pl.BlockSpec((1, ROWS_PER_BLK, W), lambda i: (i, 0, 0)),
            pl.BlockSpec((1, ROWS_PER_BLK, W), lambda i: (i, 0, 0)),
        ],
        out_shape=[
            jax.ShapeDtypeStruct((NB, ROWS_PER_BLK, W, C_OUT), f32),
            jax.ShapeDtypeStruct((NB, ROWS_PER_BLK, W, 1), f32),
            jax.ShapeDtypeStruct((NB, ROWS_PER_BLK, W), f32),
            jax.ShapeDtypeStruct((NB, ROWS_PER_BLK, W), jnp.int32),
        ],
    )(hblk, _C_UP, depth, params,
      ln_gamma.reshape(1, C_OUT), ln_beta.reshape(1, C_OUT),
      attn_w.reshape(1, C_OUT))

    feat = feat4.reshape(N_PIX, C_OUT)
    logw = logw4.reshape(N_PIX, 1)
    validf = valid4.reshape(N_PIX, 1)
    flat_idx = idx4.reshape(N_PIX)

    m = jnp.max(jnp.where(validf > 0.0, logw, -jnp.inf))

    CH = 2048
    wfeat, w = pl.pallas_call(
        _weights_kernel,
        grid=(N_PIX // CH,),
        in_specs=[
            pl.BlockSpec((CH, C_OUT), lambda i: (i, 0)),
            pl.BlockSpec((CH, 1), lambda i: (i, 0)),
            pl.BlockSpec((CH, 1), lambda i: (i, 0)),
            pl.BlockSpec((1, 1), lambda i: (0, 0)),
        ],
        out_specs=[
            pl.BlockSpec((CH, C_OUT), lambda i: (i, 0)),
            pl.BlockSpec((CH, 1), lambda i: (i, 0)),
        ],
        out_shape=[
            jax.ShapeDtypeStruct((N_PIX, C_OUT), f32),
            jax.ShapeDtypeStruct((N_PIX, 1), f32),
        ],
    )(feat, logw, validf, m.reshape(1, 1))

    acc = jnp.zeros((M_BINS, C_OUT), f32).at[flat_idx].add(wfeat)
    ws = jnp.zeros((M_BINS, 1), f32).at[flat_idx].add(w)

    RB = 8000
    bev, mask = pl.pallas_call(
        _norm_kernel,
        grid=(M_BINS // RB,),
        in_specs=[
            pl.BlockSpec((RB, C_OUT), lambda i: (i, 0)),
            pl.BlockSpec((RB, 1), lambda i: (i, 0)),
        ],
        out_specs=[
            pl.BlockSpec((RB, C_OUT), lambda i: (i, 0)),
            pl.BlockSpec((RB, 1), lambda i: (i, 0)),
        ],
        out_shape=[
            jax.ShapeDtypeStruct((M_BINS, C_OUT), f32),
            jax.ShapeDtypeStruct((M_BINS, 1), f32),
        ],
    )(acc, ws)

    bev_emb = bev.reshape(B, NX, NY, C_OUT).transpose(0, 3, 1, 2)
    bev_emb = jnp.flip(bev_emb, axis=(2, 3))
    bev_mask = mask.reshape(B, NX, NY, 1).transpose(0, 3, 1, 2)
    bev_mask = jnp.flip(bev_mask, axis=(2, 3))
    return bev_emb, bev_mask
